# baseline (speedup 1.0000x reference)
import functools

import jax
import jax.numpy as jnp
from jax.experimental import pallas as pl
from jax.experimental.pallas import tpu as pltpu

N = 4096
DIM = 512
NUM_QUANTIZERS = 12
CODEBOOK_SIZE = 1024

BN = 2048
NB = N // BN


KC = 256
NKC = CODEBOOK_SIZE // KC


def _rvq_kernel(emb_ref, cbt_ref, cb3_ref, out_ref, resid_ref, cbsq_ref):
    nb = pl.program_id(0)
    q = pl.program_id(1)

    @pl.when(q == 0)
    def _init():
        resid_ref[...] = emb_ref[...]

    r = resid_ref[...]
    cb_t = cbt_ref[0]

    @pl.when(nb == 0)
    def _cbsq():
        cbsq_ref[q, :] = jnp.sum(cb_t * cb_t, axis=0)

    cb_sq = cbsq_ref[q, :]

    dots = jax.lax.dot_general(
        r, cb_t, (((1,), (0,)), ((), ())),
        preferred_element_type=jnp.float32)
    dist = cb_sq[None, :] - 2.0 * dots
    idx = jnp.argmin(dist, axis=1).astype(jnp.int32)
    out_ref[0, 0, :] = idx

    cb = cb3_ref[0]
    mask = jnp.uint32(0xFFFF0000)
    hi32 = jax.lax.bitcast_convert_type(
        jax.lax.bitcast_convert_type(cb, jnp.uint32) & mask, jnp.float32)
    t1 = cb - hi32
    mid32 = jax.lax.bitcast_convert_type(
        jax.lax.bitcast_convert_type(t1, jnp.uint32) & mask, jnp.float32)
    hi = hi32.astype(jnp.bfloat16)
    mid = mid32.astype(jnp.bfloat16)
    lo = (t1 - mid32).astype(jnp.bfloat16)
    onehot = (jax.lax.broadcasted_iota(jnp.int32, (BN, CODEBOOK_SIZE), 1)
              == idx[:, None]).astype(jnp.bfloat16)
    dims = (((1,), (0,)), ((), ()))
    quant = (jax.lax.dot_general(onehot, hi, dims,
                                 preferred_element_type=jnp.float32)
             + jax.lax.dot_general(onehot, mid, dims,
                                   preferred_element_type=jnp.float32)
             + jax.lax.dot_general(onehot, lo, dims,
                                   preferred_element_type=jnp.float32))
    resid_ref[...] = r - quant


@functools.partial(jax.jit, static_argnames=("interpret",))
def kernel(embedding, codebooks, interpret=False):
    codebooks_t = jnp.transpose(codebooks, (0, 2, 1))
    out = pl.pallas_call(
        _rvq_kernel,
        grid=(NB, NUM_QUANTIZERS),
        in_specs=[
            pl.BlockSpec((BN, DIM), lambda nb, q: (nb, 0)),
            pl.BlockSpec((1, DIM, CODEBOOK_SIZE), lambda nb, q: (q, 0, 0)),
            pl.BlockSpec((1, CODEBOOK_SIZE, DIM), lambda nb, q: (q, 0, 0)),
        ],
        out_specs=pl.BlockSpec((1, 1, BN), lambda nb, q: (q * NB + nb, 0, 0)),
        out_shape=jax.ShapeDtypeStruct((NUM_QUANTIZERS * NB, 1, BN), jnp.int32),
        scratch_shapes=[pltpu.VMEM((BN, DIM), jnp.float32),
                        pltpu.VMEM((NUM_QUANTIZERS, CODEBOOK_SIZE),
                                   jnp.float32)],
        compiler_params=pltpu.CompilerParams(
            dimension_semantics=("arbitrary", "arbitrary")),
        interpret=interpret,
    )(embedding, codebooks_t, codebooks)
    idx = out.reshape(NUM_QUANTIZERS, N)
    return jnp.transpose(idx, (1, 0))[None]


# speedup vs baseline: 1.0446x; 1.0446x over previous
import functools

import jax
import jax.numpy as jnp
from jax.experimental import pallas as pl
from jax.experimental.pallas import tpu as pltpu

N = 4096
DIM = 512
NUM_QUANTIZERS = 12
CODEBOOK_SIZE = 1024

BN = 4096
NB = N // BN


def _rvq_kernel(emb_ref, cbt_ref, cb3_ref, out_ref, resid_ref):
    q = pl.program_id(1)

    @pl.when(q == 0)
    def _init():
        resid_ref[...] = emb_ref[...]

    r = resid_ref[...]
    cb_t = cbt_ref[0]
    cb_sq = jnp.sum(cb_t * cb_t, axis=0)
    dots = jax.lax.dot_general(
        r, cb_t, (((1,), (0,)), ((), ())),
        preferred_element_type=jnp.float32)
    dist = cb_sq[None, :] - 2.0 * dots
    idx = jnp.argmin(dist, axis=1).astype(jnp.int32)
    out_ref[0, 0, :] = idx

    cb = cb3_ref[0]
    mask = jnp.uint32(0xFFFF0000)
    hi32 = jax.lax.bitcast_convert_type(
        jax.lax.bitcast_convert_type(cb, jnp.uint32) & mask, jnp.float32)
    t1 = cb - hi32
    mid32 = jax.lax.bitcast_convert_type(
        jax.lax.bitcast_convert_type(t1, jnp.uint32) & mask, jnp.float32)
    hi = hi32.astype(jnp.bfloat16)
    mid = mid32.astype(jnp.bfloat16)
    lo = (t1 - mid32).astype(jnp.bfloat16)
    onehot = (jax.lax.broadcasted_iota(jnp.int32, (BN, CODEBOOK_SIZE), 1)
              == idx[:, None]).astype(jnp.bfloat16)
    dims = (((1,), (0,)), ((), ()))
    quant = (jax.lax.dot_general(onehot, hi, dims,
                                 preferred_element_type=jnp.float32)
             + jax.lax.dot_general(onehot, mid, dims,
                                   preferred_element_type=jnp.float32)
             + jax.lax.dot_general(onehot, lo, dims,
                                   preferred_element_type=jnp.float32))
    resid_ref[...] = r - quant


@functools.partial(jax.jit, static_argnames=("interpret",))
def kernel(embedding, codebooks, interpret=False):
    codebooks_t = jnp.transpose(codebooks, (0, 2, 1))
    out = pl.pallas_call(
        _rvq_kernel,
        grid=(NB, NUM_QUANTIZERS),
        in_specs=[
            pl.BlockSpec((BN, DIM), lambda nb, q: (nb, 0)),
            pl.BlockSpec((1, DIM, CODEBOOK_SIZE), lambda nb, q: (q, 0, 0)),
            pl.BlockSpec((1, CODEBOOK_SIZE, DIM), lambda nb, q: (q, 0, 0)),
        ],
        out_specs=pl.BlockSpec((1, 1, BN), lambda nb, q: (q * NB + nb, 0, 0)),
        out_shape=jax.ShapeDtypeStruct((NUM_QUANTIZERS * NB, 1, BN), jnp.int32),
        scratch_shapes=[pltpu.VMEM((BN, DIM), jnp.float32)],
        compiler_params=pltpu.CompilerParams(
            dimension_semantics=("arbitrary", "arbitrary")),
        interpret=interpret,
    )(embedding, codebooks_t, codebooks)
    idx = out.reshape(NUM_QUANTIZERS, N)
    return jnp.transpose(idx, (1, 0))[None]


# speedup vs baseline: 1.1758x; 1.1256x over previous
import functools

import jax
import jax.numpy as jnp
from jax.experimental import pallas as pl
from jax.experimental.pallas import tpu as pltpu

N = 4096
DIM = 512
NUM_QUANTIZERS = 12
CODEBOOK_SIZE = 1024

BN = 4096
NB = N // BN


def _rvq_kernel(emb_ref, cb_ref, out_ref, resid_ref):
    q = pl.program_id(1)

    @pl.when(q == 0)
    def _init():
        resid_ref[...] = emb_ref[...]

    r = resid_ref[...]
    cb = cb_ref[0]
    cb_t = cb.T
    cb_sq = jnp.sum(cb_t * cb_t, axis=0)
    dots = jax.lax.dot_general(
        r, cb_t, (((1,), (0,)), ((), ())),
        preferred_element_type=jnp.float32)
    dist = cb_sq[None, :] - 2.0 * dots
    idx = jnp.argmin(dist, axis=1).astype(jnp.int32)
    out_ref[0, 0, :] = idx

    mask = jnp.uint32(0xFFFF0000)
    hi32 = jax.lax.bitcast_convert_type(
        jax.lax.bitcast_convert_type(cb, jnp.uint32) & mask, jnp.float32)
    t1 = cb - hi32
    mid32 = jax.lax.bitcast_convert_type(
        jax.lax.bitcast_convert_type(t1, jnp.uint32) & mask, jnp.float32)
    hi = hi32.astype(jnp.bfloat16)
    mid = mid32.astype(jnp.bfloat16)
    lo = (t1 - mid32).astype(jnp.bfloat16)
    onehot = (jax.lax.broadcasted_iota(jnp.int32, (BN, CODEBOOK_SIZE), 1)
              == idx[:, None]).astype(jnp.bfloat16)
    dims = (((1,), (0,)), ((), ()))
    quant = (jax.lax.dot_general(onehot, hi, dims,
                                 preferred_element_type=jnp.float32)
             + jax.lax.dot_general(onehot, mid, dims,
                                   preferred_element_type=jnp.float32)
             + jax.lax.dot_general(onehot, lo, dims,
                                   preferred_element_type=jnp.float32))
    resid_ref[...] = r - quant


@functools.partial(jax.jit, static_argnames=("interpret",))
def kernel(embedding, codebooks, interpret=False):
    out = pl.pallas_call(
        _rvq_kernel,
        grid=(NB, NUM_QUANTIZERS),
        in_specs=[
            pl.BlockSpec((BN, DIM), lambda nb, q: (nb, 0)),
            pl.BlockSpec((1, CODEBOOK_SIZE, DIM), lambda nb, q: (q, 0, 0)),
        ],
        out_specs=pl.BlockSpec((1, 1, BN), lambda nb, q: (q * NB + nb, 0, 0)),
        out_shape=jax.ShapeDtypeStruct((NUM_QUANTIZERS * NB, 1, BN), jnp.int32),
        scratch_shapes=[pltpu.VMEM((BN, DIM), jnp.float32)],
        compiler_params=pltpu.CompilerParams(
            dimension_semantics=("arbitrary", "arbitrary")),
        interpret=interpret,
    )(embedding, codebooks)
    idx = out.reshape(NUM_QUANTIZERS, N)
    return jnp.transpose(idx, (1, 0))[None]
